# baseline (speedup 1.0000x reference)
import functools

import jax
import jax.numpy as jnp
import numpy as np
from jax import lax
from jax.experimental import pallas as pl
from jax.experimental.pallas import tpu as pltpu

N = 100000
E = 1600000
B = 2048
H = 32
K = 30
D = 97
DP = 128
C1 = 16
C1P = 128
TP = 15
T2 = 11
C2 = 32
F2 = T2 * C2

BBLK = 256


def _head_body(p_ref, w1_ref, m2_ref, l1_ref, l2_ref, bias_ref, o_ref):
    x = p_ref[...].reshape(BBLK * K, DP)
    y1 = jnp.maximum(jnp.dot(x, w1_ref[...],
                             preferred_element_type=jnp.float32)
                     + bias_ref[0, :C1P], 0.0)
    y1 = y1.reshape(BBLK * K // 2, 2 * C1P)
    yp = jnp.maximum(y1[:, :C1P], y1[:, C1P:])
    yp = yp.reshape(BBLK, TP * C1P)
    y2 = jnp.maximum(jnp.dot(yp, m2_ref[...],
                             preferred_element_type=jnp.float32)
                     + bias_ref[1, :F2], 0.0)
    h1 = jnp.maximum(jnp.dot(y2, l1_ref[...],
                             preferred_element_type=jnp.float32)
                     + bias_ref[2, :128], 0.0)
    o_ref[...] = (jnp.dot(h1, l2_ref[...],
                          preferred_element_type=jnp.float32)
                  + bias_ref[3, :1])


def _run_head(p_pad, conv1_w, conv1_b, conv2_w, conv2_b,
              lin1_w, lin1_b, lin2_w, lin2_b):
    w1 = jnp.zeros((DP, C1P), jnp.float32).at[:D, :C1].set(conv1_w[:, 0, :].T)
    m2 = jnp.zeros((TP * C1P, F2), jnp.float32)
    for t in range(T2):
        for k in range(5):
            m2 = m2.at[(t + k) * C1P:(t + k) * C1P + C1,
                       t * C2:(t + 1) * C2].set(conv2_w[:, :, k].T)
    b2 = jnp.tile(conv2_b[None, :], (T2, 1)).reshape(F2)
    fmap = np.arange(F2).reshape(C2, T2).T.ravel()
    l1 = lin1_w[:, fmap].T
    bias = jnp.zeros((4, F2), jnp.float32)
    bias = bias.at[0, :C1P].set(jnp.pad(conv1_b, (0, C1P - C1)))
    bias = bias.at[1, :F2].set(b2)
    bias = bias.at[2, :128].set(lin1_b)
    bias = bias.at[3, :1].set(lin2_b)
    grid = (B // BBLK,)
    return pl.pallas_call(
        _head_body,
        grid=grid,
        in_specs=[
            pl.BlockSpec((BBLK, K * DP), lambda i: (i, 0)),
            pl.BlockSpec((DP, C1P), lambda i: (0, 0)),
            pl.BlockSpec((TP * C1P, F2), lambda i: (0, 0)),
            pl.BlockSpec((F2, 128), lambda i: (0, 0)),
            pl.BlockSpec((128, 1), lambda i: (0, 0)),
            pl.BlockSpec((4, F2), lambda i: (0, 0)),
        ],
        out_specs=pl.BlockSpec((BBLK, 1), lambda i: (i, 0)),
        out_shape=jax.ShapeDtypeStruct((B, 1), jnp.float32),
    )(p_pad, w1, m2, l1, lin2_w.T, bias)


def _gcn_layer(h, src, dst, ew, W, b, dinv):
    loop = jnp.arange(N)
    s = jnp.concatenate([src, loop])
    d = jnp.concatenate([dst, loop])
    w = jnp.concatenate([ew, jnp.ones((N,), h.dtype)])
    norm = dinv[s] * w * dinv[d]
    m = h @ W
    agg = jnp.zeros((N, W.shape[1]), h.dtype).at[d].add(norm[:, None] * m[s])
    return jnp.tanh(agg + b)


def _sort_pool_pad(xc, batch):
    order = jnp.lexsort((-xc[:, -1], batch))
    xs = xc[order]
    bs = batch[order]
    counts = jnp.bincount(batch, length=B)
    starts = jnp.cumsum(counts) - counts
    rank = jnp.arange(N) - starts[bs]
    valid = rank < K
    rank_c = jnp.minimum(rank, K - 1)
    vals = jnp.where(valid[:, None], xs, 0.0)
    out = jnp.zeros((B, K, DP), jnp.float32).at[bs, rank_c, :D].add(vals)
    return out.reshape(B, K * DP)


def kernel(z, edge_index, batch, edge_weight, z_table, W0, b0, W1, b1,
           W2, b2, W3, b3, conv1_w, conv1_b, conv2_w, conv2_b,
           lin1_w, lin1_b, lin2_w, lin2_b):
    src, dst = edge_index[0], edge_index[1]
    loop = jnp.arange(N)
    d_all = jnp.concatenate([dst, loop])
    w_all = jnp.concatenate([edge_weight, jnp.ones((N,), jnp.float32)])
    deg = jnp.zeros((N,), jnp.float32).at[d_all].add(w_all)
    dinv = jnp.where(deg > 0, jax.lax.rsqrt(jnp.where(deg > 0, deg, 1.0)), 0.0)
    x = z_table[z]
    xs = []
    h = x
    for W, b in ((W0, b0), (W1, b1), (W2, b2), (W3, b3)):
        h = _gcn_layer(h, src, dst, edge_weight, W, b, dinv)
        xs.append(h)
    xc = jnp.concatenate(xs, axis=-1)
    p_pad = _sort_pool_pad(xc, batch)
    return _run_head(p_pad, conv1_w, conv1_b, conv2_w, conv2_b,
                     lin1_w, lin1_b, lin2_w, lin2_b)
